# baseline (speedup 1.0000x reference)
import functools

import jax
import jax.numpy as jnp
from jax import lax
from jax.experimental import pallas as pl
from jax.experimental.pallas import tpu as pltpu
from jax.experimental.pallas import tpu_sc as plsc

_B, _Q, _C = 8, 1024, 512
_ALPHA = 0.25
_R = _B * _Q
_INV = 1.0 / float(_R)
_SIZES = (_Q, _Q, _Q, _C, _C, _C, _C)

_NC, _NS, _L = 2, 16, 16
_NW = _NC * _NS
_RPW = _R // _NW
_NCH = _RPW // 128
_K = 7 * _NCH


def _sc_gather_body(e0, e1, e2, s0, s1, s2, s3, idx_hbm, out_hbm,
                    idx_v, flat_v, vals_v, sem):
    data = (e0, e1, e2, s0, s1, s2, s3)
    w = lax.axis_index("s") * _NC + lax.axis_index("c")
    base = w * _RPW
    pltpu.sync_copy(idx_hbm.at[w], idx_v)
    lanes = lax.iota(jnp.int32, _L)
    for k in range(_K):
        t, j = k // _NCH, k % _NCH
        n = _SIZES[t]
        for i in range(128 // _L):
            row0 = base + j * 128 + i * _L
            flat_v[k, pl.ds(i * _L, _L)] = (
                idx_v[k, pl.ds(i * _L, _L)] + (row0 + lanes) * n)
    copies = []
    for k in range(_K):
        t = k // _NCH
        copies.append(pltpu.async_copy(
            data[t].at[flat_v.at[k]], vals_v.at[k], sem))
    for c in copies:
        c.wait()
    pltpu.sync_copy(vals_v, out_hbm.at[w])


@functools.lru_cache(maxsize=None)
def _sc_gather():
    return pl.kernel(
        _sc_gather_body,
        out_type=jax.ShapeDtypeStruct((_NW, _K, 128), jnp.float32),
        mesh=plsc.VectorSubcoreMesh(core_axis_name="c", subcore_axis_name="s",
                                    num_cores=_NC, num_subcores=_NS),
        scratch_types=[
            pltpu.VMEM((_K, 128), jnp.int32),
            pltpu.VMEM((_K, 128), jnp.int32),
            pltpu.VMEM((_K, 128), jnp.float32),
            pltpu.SemaphoreType.DMA,
        ],
    )


def _fneg_sum(x):
    u = jnp.exp(-jnp.abs(x))
    sp = jnp.maximum(x, 0.0) + jnp.log1p(u)
    r = 1.0 / (1.0 + u)
    p = jnp.where(x >= 0.0, r, u * r)
    return (1.0 - _ALPHA) * jnp.sum(sp * p * p)


def _dense_body(e0, e1, e2, s0, s1, s2, s3, pp, tp, out):
    b = pl.program_id(0)
    qc = pl.program_id(1)

    @pl.when((b == 0) & (qc == 0))
    def _init():
        for t in range(8):
            out[0, t] = 0.0

    @pl.when(qc == 0)
    def _point():
        out[0, 0] += jnp.sum(jnp.abs(pp[...] - tp[...]))

    refs = (e0, e1, e2, s0, s1, s2, s3)
    for t in range(7):
        out[0, t + 1] += _fneg_sum(refs[t][...])


_QCH = 4
_dense_call = pl.pallas_call(
    _dense_body,
    grid=(_B, _QCH),
    in_specs=[
        pl.BlockSpec((1, _Q // _QCH, _Q), lambda b, qc: (b, qc, 0)),
        pl.BlockSpec((1, _Q // _QCH, _Q), lambda b, qc: (b, qc, 0)),
        pl.BlockSpec((1, _Q // _QCH, _Q), lambda b, qc: (b, qc, 0)),
        pl.BlockSpec((1, _Q // _QCH, _C), lambda b, qc: (b, qc, 0)),
        pl.BlockSpec((1, _Q // _QCH, _C), lambda b, qc: (b, qc, 0)),
        pl.BlockSpec((1, _Q // _QCH, _C), lambda b, qc: (b, qc, 0)),
        pl.BlockSpec((1, _Q // _QCH, _C), lambda b, qc: (b, qc, 0)),
        pl.BlockSpec((1, 1, 2 * _Q), lambda b, qc: (b, 0, 0)),
        pl.BlockSpec((1, 1, 2 * _Q), lambda b, qc: (b, 0, 0)),
    ],
    out_specs=pl.BlockSpec((1, 8), lambda b, qc: (0, 0),
                           memory_space=pltpu.SMEM),
    out_shape=jax.ShapeDtypeStruct((1, 8), jnp.float32),
    compiler_params=pltpu.CompilerParams(
        dimension_semantics=("arbitrary", "arbitrary")),
)


def _combine_body(part, g_ref, out):
    g = g_ref[...]
    u = jnp.exp(-jnp.abs(g))
    sp = jnp.maximum(g, 0.0) + jnp.log1p(u)
    r = 1.0 / (1.0 + u)
    p = jnp.where(g >= 0.0, r, u * r)
    q1 = 1.0 - p
    corr = _ALPHA * (sp - g) * q1 * q1 - (1.0 - _ALPHA) * sp * p * p
    out[0, 0] = part[0, 0] * _INV
    for t in range(7):
        s = jnp.sum(corr[t * 64:(t + 1) * 64, :])
        out[0, t + 1] = (part[0, t + 1] + s) * _INV


_combine_call = pl.pallas_call(
    _combine_body,
    in_specs=[
        pl.BlockSpec(memory_space=pltpu.SMEM),
        pl.BlockSpec(memory_space=pltpu.MemorySpace.VMEM),
    ],
    out_specs=pl.BlockSpec(memory_space=pltpu.SMEM),
    out_shape=jax.ShapeDtypeStruct((1, 8), jnp.float32),
)


def kernel(pred_points, pred_edges, pred_last_edges, pred_this_edges,
           pred_semantic_left_up, pred_semantic_right_up,
           pred_semantic_right_down, pred_semantic_left_down,
           target_points, edges_idx, last_edges_idx, this_edges_idx,
           sem_lu_idx, sem_ru_idx, sem_rd_idx, sem_ld_idx):
    dense = (pred_edges, pred_last_edges, pred_this_edges,
             pred_semantic_left_up, pred_semantic_right_up,
             pred_semantic_right_down, pred_semantic_left_down)
    idxs = (edges_idx, last_edges_idx, this_edges_idx,
            sem_lu_idx, sem_ru_idx, sem_rd_idx, sem_ld_idx)

    idx_all = jnp.stack([i.reshape(_R).astype(jnp.int32) for i in idxs])
    idx_w = idx_all.reshape(7, _NW, _RPW).transpose(1, 0, 2).reshape(
        _NW, _K, 128)

    flats = [d.reshape(-1) for d in dense]
    g32 = _sc_gather()(*flats, idx_w)
    g = g32.reshape(_NW, 7, _RPW).transpose(1, 0, 2).reshape(448, 128)

    part = _dense_call(*dense, target_points.reshape(_B, 1, 2 * _Q),
                       pred_points.reshape(_B, 1, 2 * _Q))
    out = _combine_call(part, g)
    return out.reshape(8)


# speedup vs baseline: 2.5506x; 2.5506x over previous
import functools

import jax
import jax.numpy as jnp
from jax import lax
from jax.experimental import pallas as pl
from jax.experimental.pallas import tpu as pltpu
from jax.experimental.pallas import tpu_sc as plsc

_B, _Q, _C = 8, 1024, 512
_ALPHA = 0.25
_R = _B * _Q
_INV = 1.0 / float(_R)
_SIZES = (_Q, _Q, _Q, _C, _C, _C, _C)

_NC, _NS, _L = 2, 16, 16
_NW = _NC * _NS
_RPW = _R // _NW
_FPW = 7 * _RPW
_RPC = 32
_CHUNKS = tuple((t, j) for t in range(7) for j in range(_RPW // _RPC))


def _sc_extract_body(e0, e1, e2, s0, s1, s2, s3, idx_hbm, out_hbm,
                     idx_v, vals_v, bufq, bufc, sem0, sem1):
    data = (e0, e1, e2, s0, s1, s2, s3)
    sems = (sem0, sem1)
    w = lax.axis_index("s") * _NC + lax.axis_index("c")
    base = w * _RPW
    off = w * _FPW
    pltpu.sync_copy(idx_hbm.at[pl.ds(off, _FPW)], idx_v)
    lanes = lax.iota(jnp.int32, _L)

    def src(c):
        t, j = _CHUNKS[c]
        return data[t].at[pl.ds(base + j * _RPC, _RPC), :]

    def buf(c):
        t, _ = _CHUNKS[c]
        return (bufq if t < 3 else bufc).at[c & 1]

    n = len(_CHUNKS)
    pend = {0: pltpu.async_copy(src(0), buf(0), sems[0])}
    for c in range(n):
        if c + 1 < n:
            pend[c + 1] = pltpu.async_copy(src(c + 1), buf(c + 1),
                                           sems[(c + 1) & 1])
        pend[c].wait()
        t, j = _CHUNKS[c]
        nn = _SIZES[t]
        b = buf(c)
        for i in range(_RPC // _L):
            pos = t * _RPW + j * _RPC + i * _L
            cols = idx_v[pl.ds(pos, _L)]
            lr = i * _L + lanes
            raw = ((lr >> 3) * (8 * nn) + (cols >> 7) * 1024
                   + (lr & 7) * 128 + (cols & 127))
            sh = 10 if nn == _Q else 9
            vals_v[pl.ds(pos, _L)] = plsc.load_gather(
                b, [raw >> sh, raw & (nn - 1)])
    pltpu.sync_copy(vals_v, out_hbm.at[pl.ds(off, _FPW)])


@functools.lru_cache(maxsize=None)
def _sc_extract():
    return pl.kernel(
        _sc_extract_body,
        out_type=jax.ShapeDtypeStruct((_NW * _FPW,), jnp.float32),
        mesh=plsc.VectorSubcoreMesh(core_axis_name="c", subcore_axis_name="s",
                                    num_cores=_NC, num_subcores=_NS),
        scratch_types=[
            pltpu.VMEM((_FPW,), jnp.int32),
            pltpu.VMEM((_FPW,), jnp.float32),
            pltpu.VMEM((2, _RPC, _Q), jnp.float32),
            pltpu.VMEM((2, _RPC, _C), jnp.float32),
            pltpu.SemaphoreType.DMA,
            pltpu.SemaphoreType.DMA,
        ],
        compiler_params=pltpu.CompilerParams(use_tc_tiling_on_sc=True,
                                             needs_layout_passes=False),
    )


def _fneg_sum(x):
    th = jnp.tanh(0.5 * x)
    p = 0.5 * (1.0 + th)
    q = 0.5 * (1.0 - th)
    return -(1.0 - _ALPHA) * jnp.sum(jnp.log(q) * p * p)


def _dense_body(e0, e1, e2, s0, s1, s2, s3, pp, tp, out):
    b = pl.program_id(0)
    qc = pl.program_id(1)

    @pl.when((b == 0) & (qc == 0))
    def _init():
        for t in range(8):
            out[0, t] = 0.0

    @pl.when(qc == 0)
    def _point():
        out[0, 0] += jnp.sum(jnp.abs(pp[...] - tp[...]))

    refs = (e0, e1, e2, s0, s1, s2, s3)
    for t in range(7):
        out[0, t + 1] += _fneg_sum(refs[t][...])


_QCH = 4
_dense_call = pl.pallas_call(
    _dense_body,
    grid=(_B, _QCH),
    in_specs=[
        pl.BlockSpec((1, _Q // _QCH, _Q), lambda b, qc: (b, qc, 0)),
        pl.BlockSpec((1, _Q // _QCH, _Q), lambda b, qc: (b, qc, 0)),
        pl.BlockSpec((1, _Q // _QCH, _Q), lambda b, qc: (b, qc, 0)),
        pl.BlockSpec((1, _Q // _QCH, _C), lambda b, qc: (b, qc, 0)),
        pl.BlockSpec((1, _Q // _QCH, _C), lambda b, qc: (b, qc, 0)),
        pl.BlockSpec((1, _Q // _QCH, _C), lambda b, qc: (b, qc, 0)),
        pl.BlockSpec((1, _Q // _QCH, _C), lambda b, qc: (b, qc, 0)),
        pl.BlockSpec((1, 1, 2 * _Q), lambda b, qc: (b, 0, 0)),
        pl.BlockSpec((1, 1, 2 * _Q), lambda b, qc: (b, 0, 0)),
    ],
    out_specs=pl.BlockSpec((1, 8), lambda b, qc: (0, 0),
                           memory_space=pltpu.SMEM),
    out_shape=jax.ShapeDtypeStruct((1, 8), jnp.float32),
    compiler_params=pltpu.CompilerParams(
        dimension_semantics=("arbitrary", "arbitrary")),
)


def _combine_body(part, g_ref, out):
    g = g_ref[...]
    th = jnp.tanh(0.5 * g)
    p = 0.5 * (1.0 + th)
    q = 0.5 * (1.0 - th)
    corr = (1.0 - _ALPHA) * jnp.log(q) * p * p - _ALPHA * jnp.log(p) * q * q
    out[0, 0] = part[0, 0] * _INV
    for t in range(7):
        s = jnp.sum(corr[t * 64:(t + 1) * 64, :])
        out[0, t + 1] = (part[0, t + 1] + s) * _INV


_combine_call = pl.pallas_call(
    _combine_body,
    in_specs=[
        pl.BlockSpec(memory_space=pltpu.SMEM),
        pl.BlockSpec(memory_space=pltpu.MemorySpace.VMEM),
    ],
    out_specs=pl.BlockSpec(memory_space=pltpu.SMEM),
    out_shape=jax.ShapeDtypeStruct((1, 8), jnp.float32),
)


def kernel(pred_points, pred_edges, pred_last_edges, pred_this_edges,
           pred_semantic_left_up, pred_semantic_right_up,
           pred_semantic_right_down, pred_semantic_left_down,
           target_points, edges_idx, last_edges_idx, this_edges_idx,
           sem_lu_idx, sem_ru_idx, sem_rd_idx, sem_ld_idx):
    dense = (pred_edges, pred_last_edges, pred_this_edges,
             pred_semantic_left_up, pred_semantic_right_up,
             pred_semantic_right_down, pred_semantic_left_down)
    idxs = (edges_idx, last_edges_idx, this_edges_idx,
            sem_lu_idx, sem_ru_idx, sem_rd_idx, sem_ld_idx)

    idx_all = jnp.stack([i.reshape(_R).astype(jnp.int32) for i in idxs])
    idx_flat = idx_all.reshape(7, _NW, _RPW).transpose(1, 0, 2).reshape(-1)

    rows2d = [d.reshape(_R, n) for d, n in zip(dense, _SIZES)]
    gflat = _sc_extract()(*rows2d, idx_flat)
    g = gflat.reshape(_NW, 7, _RPW).transpose(1, 0, 2).reshape(448, 128)

    part = _dense_call(*dense, target_points.reshape(_B, 1, 2 * _Q),
                       pred_points.reshape(_B, 1, 2 * _Q))
    out = _combine_call(part, g)
    return out.reshape(8)


# speedup vs baseline: 3.0943x; 1.2132x over previous
import functools

import jax
import jax.numpy as jnp
from jax import lax
from jax.experimental import pallas as pl
from jax.experimental.pallas import tpu as pltpu
from jax.experimental.pallas import tpu_sc as plsc

_B, _Q, _C = 8, 1024, 512
_ALPHA = 0.25
_R = _B * _Q
_INV = 1.0 / float(_R)
_SIZES = (_Q, _Q, _Q, _C, _C, _C, _C)

_NC, _NS, _L = 2, 16, 16
_NW = _NC * _NS
_RPW = _R // _NW
_FPW = 7 * _RPW
_GK = _RPW // 8
_GROUPS = tuple((t, sl) for t in range(7) for sl in range(8))


def _sc_extract_body(e0, e1, e2, s0, s1, s2, s3, idx_hbm, out_hbm,
                     idx_v, vals_v, midx_v, lane_v, buf, sem0, sem1):
    data = (e0, e1, e2, s0, s1, s2, s3)
    sems = (sem0, sem1)
    w = lax.axis_index("s") * _NC + lax.axis_index("c")
    base = w * _RPW
    off = w * _FPW
    pltpu.sync_copy(idx_hbm.at[pl.ds(off, _FPW)], idx_v)
    lanes = lax.iota(jnp.int32, _L)

    def prep(g, s):
        t, sl = _GROUPS[g]
        for i in range(_GK // _L):
            pos = (t * _RPW + sl + 128 * i) + 8 * lanes
            cols = plsc.load_gather(idx_v, [pos])
            k = i * _L + lanes
            ct = cols >> 7
            if _SIZES[t] == _Q:
                m = base + 8 * k + ct
            else:
                m = base + 8 * k + 2 * ct + (sl >> 2)
            midx_v[s, pl.ds(i * _L, _L)] = m
            lane_v[s, pl.ds(i * _L, _L)] = cols & 127

    def fire(g, s):
        t, sl = _GROUPS[g]
        c0 = (sl if _SIZES[t] == _Q else (sl & 3)) * 128
        return pltpu.async_copy(
            data[t].at[:, pl.ds(c0, 128)].at[midx_v.at[s]],
            buf.at[s], sems[s])

    n = len(_GROUPS)
    prep(0, 0)
    pend = {0: fire(0, 0)}
    for g in range(n):
        s = g & 1
        if g + 1 < n:
            prep(g + 1, 1 - s)
            pend[g + 1] = fire(g + 1, 1 - s)
        pend[g].wait()
        t, sl = _GROUPS[g]
        for i in range(_GK // _L):
            k = i * _L + lanes
            vals = plsc.load_gather(
                buf.at[s], [k, lane_v[s, pl.ds(i * _L, _L)]])
            pos = (t * _RPW + sl + 128 * i) + 8 * lanes
            plsc.store_scatter(vals_v, [pos], vals)
    pltpu.sync_copy(vals_v, out_hbm.at[pl.ds(off, _FPW)])


@functools.lru_cache(maxsize=None)
def _sc_extract():
    return pl.kernel(
        _sc_extract_body,
        out_type=jax.ShapeDtypeStruct((_NW * _FPW,), jnp.float32),
        mesh=plsc.VectorSubcoreMesh(core_axis_name="c", subcore_axis_name="s",
                                    num_cores=_NC, num_subcores=_NS),
        scratch_types=[
            pltpu.VMEM((_FPW,), jnp.int32),
            pltpu.VMEM((_FPW,), jnp.float32),
            pltpu.VMEM((2, _GK), jnp.int32),
            pltpu.VMEM((2, _GK), jnp.int32),
            pltpu.VMEM((2, _GK, 128), jnp.float32),
            pltpu.SemaphoreType.DMA,
            pltpu.SemaphoreType.DMA,
        ],
        compiler_params=pltpu.CompilerParams(use_tc_tiling_on_sc=True,
                                             needs_layout_passes=False),
    )


def _fneg_sum(x):
    th = jnp.tanh(0.5 * x)
    p = 0.5 * (1.0 + th)
    q = 0.5 * (1.0 - th)
    return -(1.0 - _ALPHA) * jnp.sum(jnp.log(q) * p * p)


def _dense_body(e0, e1, e2, s0, s1, s2, s3, pp, tp, out):
    b = pl.program_id(0)
    qc = pl.program_id(1)

    @pl.when((b == 0) & (qc == 0))
    def _init():
        for t in range(8):
            out[0, t] = 0.0

    @pl.when(qc == 0)
    def _point():
        out[0, 0] += jnp.sum(jnp.abs(pp[...] - tp[...]))

    refs = (e0, e1, e2, s0, s1, s2, s3)
    for t in range(7):
        out[0, t + 1] += _fneg_sum(refs[t][...])


_QCH = 4
_dense_call = pl.pallas_call(
    _dense_body,
    grid=(_B, _QCH),
    in_specs=[
        pl.BlockSpec((1, _Q // _QCH, _Q), lambda b, qc: (b, qc, 0)),
        pl.BlockSpec((1, _Q // _QCH, _Q), lambda b, qc: (b, qc, 0)),
        pl.BlockSpec((1, _Q // _QCH, _Q), lambda b, qc: (b, qc, 0)),
        pl.BlockSpec((1, _Q // _QCH, _C), lambda b, qc: (b, qc, 0)),
        pl.BlockSpec((1, _Q // _QCH, _C), lambda b, qc: (b, qc, 0)),
        pl.BlockSpec((1, _Q // _QCH, _C), lambda b, qc: (b, qc, 0)),
        pl.BlockSpec((1, _Q // _QCH, _C), lambda b, qc: (b, qc, 0)),
        pl.BlockSpec((1, 1, 2 * _Q), lambda b, qc: (b, 0, 0)),
        pl.BlockSpec((1, 1, 2 * _Q), lambda b, qc: (b, 0, 0)),
    ],
    out_specs=pl.BlockSpec((1, 8), lambda b, qc: (0, 0),
                           memory_space=pltpu.SMEM),
    out_shape=jax.ShapeDtypeStruct((1, 8), jnp.float32),
    compiler_params=pltpu.CompilerParams(
        dimension_semantics=("arbitrary", "arbitrary")),
)


def _combine_body(part, g_ref, out):
    g = g_ref[...]
    th = jnp.tanh(0.5 * g)
    p = 0.5 * (1.0 + th)
    q = 0.5 * (1.0 - th)
    corr = (1.0 - _ALPHA) * jnp.log(q) * p * p - _ALPHA * jnp.log(p) * q * q
    out[0, 0] = part[0, 0] * _INV
    for t in range(7):
        s = jnp.sum(corr[t * 64:(t + 1) * 64, :])
        out[0, t + 1] = (part[0, t + 1] + s) * _INV


_combine_call = pl.pallas_call(
    _combine_body,
    in_specs=[
        pl.BlockSpec(memory_space=pltpu.SMEM),
        pl.BlockSpec(memory_space=pltpu.MemorySpace.VMEM),
    ],
    out_specs=pl.BlockSpec(memory_space=pltpu.SMEM),
    out_shape=jax.ShapeDtypeStruct((1, 8), jnp.float32),
)


def kernel(pred_points, pred_edges, pred_last_edges, pred_this_edges,
           pred_semantic_left_up, pred_semantic_right_up,
           pred_semantic_right_down, pred_semantic_left_down,
           target_points, edges_idx, last_edges_idx, this_edges_idx,
           sem_lu_idx, sem_ru_idx, sem_rd_idx, sem_ld_idx):
    dense = (pred_edges, pred_last_edges, pred_this_edges,
             pred_semantic_left_up, pred_semantic_right_up,
             pred_semantic_right_down, pred_semantic_left_down)
    idxs = (edges_idx, last_edges_idx, this_edges_idx,
            sem_lu_idx, sem_ru_idx, sem_rd_idx, sem_ld_idx)

    idx_all = jnp.stack([i.reshape(_R).astype(jnp.int32) for i in idxs])
    idx_flat = idx_all.reshape(7, _NW, _RPW).transpose(1, 0, 2).reshape(-1)

    rows2d = [d.reshape(_R, n) for d, n in zip(dense, _SIZES)]
    gflat = _sc_extract()(*rows2d, idx_flat)
    g = gflat.reshape(_NW, 7, _RPW).transpose(1, 0, 2).reshape(448, 128)

    part = _dense_call(*dense, target_points.reshape(_B, 1, 2 * _Q),
                       pred_points.reshape(_B, 1, 2 * _Q))
    out = _combine_call(part, g)
    return out.reshape(8)


# speedup vs baseline: 3.2396x; 1.0469x over previous
import functools

import jax
import jax.numpy as jnp
from jax import lax
from jax.experimental import pallas as pl
from jax.experimental.pallas import tpu as pltpu
from jax.experimental.pallas import tpu_sc as plsc

_B, _Q, _C = 8, 1024, 512
_ALPHA = 0.25
_R = _B * _Q
_INV = 1.0 / float(_R)
_SIZES = (_Q, _Q, _Q, _C, _C, _C, _C)

_NC, _NS, _L = 2, 16, 16
_NW = _NC * _NS
_RPW = _R // _NW
_FPW = 7 * _RPW
_GK = _RPW // 8
_GROUPS = tuple((t, sl) for t in range(7) for sl in range(8))


def _sc_extract_body(e0, e1, e2, s0, s1, s2, s3, idx_hbm, out_hbm,
                     idx_v, vals_v, midx_v, lane_v, buf, sem0, sem1):
    data = (e0, e1, e2, s0, s1, s2, s3)
    sems = (sem0, sem1)
    w = lax.axis_index("s") * _NC + lax.axis_index("c")
    base = w * _RPW
    off = w * _FPW
    pltpu.sync_copy(idx_hbm.at[pl.ds(off, _FPW)], idx_v)
    lanes = lax.iota(jnp.int32, _L)

    def prep(g, s):
        t, sl = _GROUPS[g]
        for i in range(_GK // _L):
            pos = (t * _RPW + sl + 128 * i) + 8 * lanes
            cols = plsc.load_gather(idx_v, [pos])
            k = i * _L + lanes
            ct = cols >> 7
            if _SIZES[t] == _Q:
                m = base + 8 * k + ct
            else:
                m = base + 8 * k + 2 * ct + (sl >> 2)
            midx_v[s, pl.ds(i * _L, _L)] = m
            lane_v[s, pl.ds(i * _L, _L)] = cols & 127

    def fire(g, s):
        t, sl = _GROUPS[g]
        c0 = (sl if _SIZES[t] == _Q else (sl & 3)) * 128
        return pltpu.async_copy(
            data[t].at[:, pl.ds(c0, 128)].at[midx_v.at[s]],
            buf.at[s], sems[s])

    n = len(_GROUPS)
    prep(0, 0)
    pend = {0: fire(0, 0)}
    for g in range(n):
        s = g & 1
        if g + 1 < n:
            prep(g + 1, 1 - s)
            pend[g + 1] = fire(g + 1, 1 - s)
        pend[g].wait()
        t, sl = _GROUPS[g]
        for i in range(_GK // _L):
            k = i * _L + lanes
            vals = plsc.load_gather(
                buf.at[s], [k, lane_v[s, pl.ds(i * _L, _L)]])
            pos = (t * _RPW + sl + 128 * i) + 8 * lanes
            plsc.store_scatter(vals_v, [pos], vals)
    pltpu.sync_copy(vals_v, out_hbm.at[pl.ds(off, _FPW)])


@functools.lru_cache(maxsize=None)
def _sc_extract():
    return pl.kernel(
        _sc_extract_body,
        out_type=jax.ShapeDtypeStruct((_NW * _FPW,), jnp.float32),
        mesh=plsc.VectorSubcoreMesh(core_axis_name="c", subcore_axis_name="s",
                                    num_cores=_NC, num_subcores=_NS),
        scratch_types=[
            pltpu.VMEM((_FPW,), jnp.int32),
            pltpu.VMEM((_FPW,), jnp.float32),
            pltpu.VMEM((2, _GK), jnp.int32),
            pltpu.VMEM((2, _GK), jnp.int32),
            pltpu.VMEM((2, _GK, 128), jnp.float32),
            pltpu.SemaphoreType.DMA,
            pltpu.SemaphoreType.DMA,
        ],
        compiler_params=pltpu.CompilerParams(use_tc_tiling_on_sc=True,
                                             needs_layout_passes=False),
    )


_LN2 = 0.6931471805599453
_CNEG = -(1.0 - _ALPHA) * 0.25


_CH = 16


def _fneg_sum(ref):
    n = ref.shape[2]
    acc = jnp.zeros((8, n), jnp.float32)
    for c in range(ref.shape[1] // _CH):
        x = ref[0, c * _CH:(c + 1) * _CH, :]
        t = jnp.tanh(0.5 * x)
        s = 1.0 + t
        v = (jnp.log(1.0 - t) - _LN2) * s * s
        acc += v.reshape(_CH // 8, 8, n).sum(axis=0)
    return jnp.sum(acc)


def _dense_body(e0, e1, e2, s0, s1, s2, s3, pp, tp, out):
    b = pl.program_id(0)
    qc = pl.program_id(1)

    @pl.when((b == 0) & (qc == 0))
    def _init():
        for t in range(8):
            out[0, t] = 0.0

    @pl.when(qc == 0)
    def _point():
        out[0, 0] += jnp.sum(jnp.abs(pp[...] - tp[...]))

    refs = (e0, e1, e2, s0, s1, s2, s3)
    for t in range(7):
        out[0, t + 1] += _fneg_sum(refs[t])


_QCH = 4
_dense_call = pl.pallas_call(
    _dense_body,
    grid=(_B, _QCH),
    in_specs=[
        pl.BlockSpec((1, _Q // _QCH, _Q), lambda b, qc: (b, qc, 0)),
        pl.BlockSpec((1, _Q // _QCH, _Q), lambda b, qc: (b, qc, 0)),
        pl.BlockSpec((1, _Q // _QCH, _Q), lambda b, qc: (b, qc, 0)),
        pl.BlockSpec((1, _Q // _QCH, _C), lambda b, qc: (b, qc, 0)),
        pl.BlockSpec((1, _Q // _QCH, _C), lambda b, qc: (b, qc, 0)),
        pl.BlockSpec((1, _Q // _QCH, _C), lambda b, qc: (b, qc, 0)),
        pl.BlockSpec((1, _Q // _QCH, _C), lambda b, qc: (b, qc, 0)),
        pl.BlockSpec((1, 1, 2 * _Q), lambda b, qc: (b, 0, 0)),
        pl.BlockSpec((1, 1, 2 * _Q), lambda b, qc: (b, 0, 0)),
    ],
    out_specs=pl.BlockSpec((1, 8), lambda b, qc: (0, 0),
                           memory_space=pltpu.SMEM),
    out_shape=jax.ShapeDtypeStruct((1, 8), jnp.float32),
    compiler_params=pltpu.CompilerParams(
        dimension_semantics=("arbitrary", "arbitrary")),
)


def _combine_body(part, g_ref, out):
    g = g_ref[...]
    th = jnp.tanh(0.5 * g)
    p = 0.5 * (1.0 + th)
    q = 0.5 * (1.0 - th)
    corr = (1.0 - _ALPHA) * jnp.log(q) * p * p - _ALPHA * jnp.log(p) * q * q
    out[0, 0] = part[0, 0] * _INV
    for t in range(7):
        s = jnp.sum(corr[t * 64:(t + 1) * 64, :])
        out[0, t + 1] = (_CNEG * part[0, t + 1] + s) * _INV


_combine_call = pl.pallas_call(
    _combine_body,
    in_specs=[
        pl.BlockSpec(memory_space=pltpu.SMEM),
        pl.BlockSpec(memory_space=pltpu.MemorySpace.VMEM),
    ],
    out_specs=pl.BlockSpec(memory_space=pltpu.SMEM),
    out_shape=jax.ShapeDtypeStruct((1, 8), jnp.float32),
)


def kernel(pred_points, pred_edges, pred_last_edges, pred_this_edges,
           pred_semantic_left_up, pred_semantic_right_up,
           pred_semantic_right_down, pred_semantic_left_down,
           target_points, edges_idx, last_edges_idx, this_edges_idx,
           sem_lu_idx, sem_ru_idx, sem_rd_idx, sem_ld_idx):
    dense = (pred_edges, pred_last_edges, pred_this_edges,
             pred_semantic_left_up, pred_semantic_right_up,
             pred_semantic_right_down, pred_semantic_left_down)
    idxs = (edges_idx, last_edges_idx, this_edges_idx,
            sem_lu_idx, sem_ru_idx, sem_rd_idx, sem_ld_idx)

    idx_all = jnp.stack([i.reshape(_R).astype(jnp.int32) for i in idxs])
    idx_flat = idx_all.reshape(7, _NW, _RPW).transpose(1, 0, 2).reshape(-1)

    rows2d = [d.reshape(_R, n) for d, n in zip(dense, _SIZES)]
    gflat = _sc_extract()(*rows2d, idx_flat)
    g = gflat.reshape(_NW, 7, _RPW).transpose(1, 0, 2).reshape(448, 128)

    part = _dense_call(*dense, target_points.reshape(_B, 1, 2 * _Q),
                       pred_points.reshape(_B, 1, 2 * _Q))
    out = _combine_call(part, g)
    return out.reshape(8)


# speedup vs baseline: 3.5450x; 1.0943x over previous
import functools

import jax
import jax.numpy as jnp
from jax import lax
from jax.experimental import pallas as pl
from jax.experimental.pallas import tpu as pltpu
from jax.experimental.pallas import tpu_sc as plsc

_B, _Q, _C = 8, 1024, 512
_ALPHA = 0.25
_R = _B * _Q
_INV = 1.0 / float(_R)
_SIZES = (_Q, _Q, _Q, _C, _C, _C, _C)

_NC, _NS, _L = 2, 16, 16
_NW = _NC * _NS
_RPW = _R // _NW
_FPW = 7 * _RPW
_GK = _RPW // 8
_GROUPS = tuple((t, sl) for t in range(7) for sl in range(8))


def _sc_extract_body(e0, e1, e2, s0, s1, s2, s3, idx_hbm, out_hbm,
                     idx_v, vals_v, midx_v, lane_v, buf, sem0, sem1):
    data = (e0, e1, e2, s0, s1, s2, s3)
    sems = (sem0, sem1)
    w = lax.axis_index("s") * _NC + lax.axis_index("c")
    base = w * _RPW
    off = w * _FPW
    pltpu.sync_copy(idx_hbm.at[pl.ds(off, _FPW)], idx_v)
    lanes = lax.iota(jnp.int32, _L)

    def prep(g, s):
        t, sl = _GROUPS[g]
        for i in range(_GK // _L):
            pos = (t * _RPW + sl + 128 * i) + 8 * lanes
            cols = plsc.load_gather(idx_v, [pos])
            k = i * _L + lanes
            ct = cols >> 7
            if _SIZES[t] == _Q:
                m = base + 8 * k + ct
            else:
                m = base + 8 * k + 2 * ct + (sl >> 2)
            midx_v[s, pl.ds(i * _L, _L)] = m
            lane_v[s, pl.ds(i * _L, _L)] = cols & 127

    def fire(g, s):
        t, sl = _GROUPS[g]
        c0 = (sl if _SIZES[t] == _Q else (sl & 3)) * 128
        return pltpu.async_copy(
            data[t].at[:, pl.ds(c0, 128)].at[midx_v.at[s]],
            buf.at[s], sems[s])

    n = len(_GROUPS)
    prep(0, 0)
    pend = {0: fire(0, 0)}
    for g in range(n):
        s = g & 1
        if g + 1 < n:
            prep(g + 1, 1 - s)
            pend[g + 1] = fire(g + 1, 1 - s)
        pend[g].wait()
        t, sl = _GROUPS[g]
        for i in range(_GK // _L):
            k = i * _L + lanes
            vals = plsc.load_gather(
                buf.at[s], [k, lane_v[s, pl.ds(i * _L, _L)]])
            pos = (t * _RPW + sl + 128 * i) + 8 * lanes
            plsc.store_scatter(vals_v, [pos], vals)
    pltpu.sync_copy(vals_v, out_hbm.at[pl.ds(off, _FPW)])


@functools.lru_cache(maxsize=None)
def _sc_extract():
    return pl.kernel(
        _sc_extract_body,
        out_type=jax.ShapeDtypeStruct((_NW * _FPW,), jnp.float32),
        mesh=plsc.VectorSubcoreMesh(core_axis_name="c", subcore_axis_name="s",
                                    num_cores=_NC, num_subcores=_NS),
        scratch_types=[
            pltpu.VMEM((_FPW,), jnp.int32),
            pltpu.VMEM((_FPW,), jnp.float32),
            pltpu.VMEM((2, _GK), jnp.int32),
            pltpu.VMEM((2, _GK), jnp.int32),
            pltpu.VMEM((2, _GK, 128), jnp.float32),
            pltpu.SemaphoreType.DMA,
            pltpu.SemaphoreType.DMA,
        ],
        compiler_params=pltpu.CompilerParams(use_tc_tiling_on_sc=True,
                                             needs_layout_passes=False),
    )


_LN2 = 0.6931471805599453
_CNEG = -(1.0 - _ALPHA) * 0.25


_CH = 16


def _fneg_sum(ref):
    n = ref.shape[2]
    acc = jnp.zeros((8, n), jnp.float32)
    for c in range(ref.shape[1] // _CH):
        x = ref[0, c * _CH:(c + 1) * _CH, :]
        t = jnp.tanh(0.5 * x)
        s = 1.0 + t
        v = (jnp.log(1.0 - t) - _LN2) * s * s
        acc += v.reshape(_CH // 8, 8, n).sum(axis=0)
    return jnp.sum(acc)


def _dense_body(e0, e1, e2, s0, s1, s2, s3, pp, tp, out):
    b = pl.program_id(0)
    qc = pl.program_id(1)

    @pl.when((b == 0) & (qc == 0))
    def _init():
        for t in range(8):
            out[0, t] = 0.0

    @pl.when(qc == 0)
    def _point():
        out[0, 0] += jnp.sum(jnp.abs(pp[...] - tp[...]))

    refs = (e0, e1, e2, s0, s1, s2, s3)
    for t in range(7):
        out[0, t + 1] += _fneg_sum(refs[t])


_QCH = 1
_dense_call = pl.pallas_call(
    _dense_body,
    grid=(_B, _QCH),
    in_specs=[
        pl.BlockSpec((1, _Q // _QCH, _Q), lambda b, qc: (b, qc, 0)),
        pl.BlockSpec((1, _Q // _QCH, _Q), lambda b, qc: (b, qc, 0)),
        pl.BlockSpec((1, _Q // _QCH, _Q), lambda b, qc: (b, qc, 0)),
        pl.BlockSpec((1, _Q // _QCH, _C), lambda b, qc: (b, qc, 0)),
        pl.BlockSpec((1, _Q // _QCH, _C), lambda b, qc: (b, qc, 0)),
        pl.BlockSpec((1, _Q // _QCH, _C), lambda b, qc: (b, qc, 0)),
        pl.BlockSpec((1, _Q // _QCH, _C), lambda b, qc: (b, qc, 0)),
        pl.BlockSpec((1, 1, 2 * _Q), lambda b, qc: (b, 0, 0)),
        pl.BlockSpec((1, 1, 2 * _Q), lambda b, qc: (b, 0, 0)),
    ],
    out_specs=pl.BlockSpec((1, 8), lambda b, qc: (0, 0),
                           memory_space=pltpu.SMEM),
    out_shape=jax.ShapeDtypeStruct((1, 8), jnp.float32),
    compiler_params=pltpu.CompilerParams(
        dimension_semantics=("arbitrary", "arbitrary")),
)


def _combine_body(part, g_ref, out):
    g = g_ref[...]
    th = jnp.tanh(0.5 * g)
    p = 0.5 * (1.0 + th)
    q = 0.5 * (1.0 - th)
    corr = (1.0 - _ALPHA) * jnp.log(q) * p * p - _ALPHA * jnp.log(p) * q * q
    out[0, 0] = part[0, 0] * _INV
    for t in range(7):
        s = jnp.sum(corr[t * 64:(t + 1) * 64, :])
        out[0, t + 1] = (_CNEG * part[0, t + 1] + s) * _INV


_combine_call = pl.pallas_call(
    _combine_body,
    in_specs=[
        pl.BlockSpec(memory_space=pltpu.SMEM),
        pl.BlockSpec(memory_space=pltpu.MemorySpace.VMEM),
    ],
    out_specs=pl.BlockSpec(memory_space=pltpu.SMEM),
    out_shape=jax.ShapeDtypeStruct((1, 8), jnp.float32),
)


def kernel(pred_points, pred_edges, pred_last_edges, pred_this_edges,
           pred_semantic_left_up, pred_semantic_right_up,
           pred_semantic_right_down, pred_semantic_left_down,
           target_points, edges_idx, last_edges_idx, this_edges_idx,
           sem_lu_idx, sem_ru_idx, sem_rd_idx, sem_ld_idx):
    dense = (pred_edges, pred_last_edges, pred_this_edges,
             pred_semantic_left_up, pred_semantic_right_up,
             pred_semantic_right_down, pred_semantic_left_down)
    idxs = (edges_idx, last_edges_idx, this_edges_idx,
            sem_lu_idx, sem_ru_idx, sem_rd_idx, sem_ld_idx)

    idx_all = jnp.stack([i.reshape(_R).astype(jnp.int32) for i in idxs])
    idx_flat = idx_all.reshape(7, _NW, _RPW).transpose(1, 0, 2).reshape(-1)

    rows2d = [d.reshape(_R, n) for d, n in zip(dense, _SIZES)]
    gflat = _sc_extract()(*rows2d, idx_flat)
    g = gflat.reshape(_NW, 7, _RPW).transpose(1, 0, 2).reshape(448, 128)

    part = _dense_call(*dense, target_points.reshape(_B, 1, 2 * _Q),
                       pred_points.reshape(_B, 1, 2 * _Q))
    out = _combine_call(part, g)
    return out.reshape(8)


# speedup vs baseline: 3.6694x; 1.0351x over previous
import functools

import jax
import jax.numpy as jnp
from jax import lax
from jax.experimental import pallas as pl
from jax.experimental.pallas import tpu as pltpu
from jax.experimental.pallas import tpu_sc as plsc

_B, _Q, _C = 8, 1024, 512
_ALPHA = 0.25
_R = _B * _Q
_INV = 1.0 / float(_R)
_SIZES = (_Q, _Q, _Q, _C, _C, _C, _C)

_NC, _NS, _L = 2, 16, 16
_NW = _NC * _NS
_RPW = _R // _NW
_FPW = 7 * _RPW
_GK = _RPW // 8
_GROUPS = tuple((t, sl) for t in range(7) for sl in range(8))


def _sc_extract_body(e0, e1, e2, s0, s1, s2, s3,
                     i0, i1, i2, i3, i4, i5, i6, out_hbm,
                     idxb, vals_v, midx_v, lane_v, buf, sem0, sem1):
    data = (e0, e1, e2, s0, s1, s2, s3)
    idxs = (i0, i1, i2, i3, i4, i5, i6)
    sems = (sem0, sem1)
    w = lax.axis_index("s") * _NC + lax.axis_index("c")
    b = w // 4
    qq = w % 4
    base = w * _RPW
    lanes = lax.iota(jnp.int32, _L)

    ih = [pltpu.async_copy(
              idxs[t].at[:, pl.ds(b * 128, 128)].at[2 * qq + j],
              idxb.at[t].at[j], sems[j])
          for t in range(7) for j in range(2)]
    idx_ready = [False] * 7

    def prep(g, s):
        t, sl = _GROUPS[g]
        if not idx_ready[t]:
            ih[2 * t].wait()
            ih[2 * t + 1].wait()
            idx_ready[t] = True
        for i in range(_GK // _L):
            cols = plsc.load_gather(idxb.at[t].at[i], [sl + 8 * lanes])
            k = i * _L + lanes
            ct = cols >> 7
            if _SIZES[t] == _Q:
                m = base + 8 * k + ct
            else:
                m = base + 8 * k + 2 * ct + (sl >> 2)
            midx_v[s, pl.ds(i * _L, _L)] = m
            lane_v[s, pl.ds(i * _L, _L)] = cols & 127

    def fire(g, s):
        t, sl = _GROUPS[g]
        c0 = (sl if _SIZES[t] == _Q else (sl & 3)) * 128
        return pltpu.async_copy(
            data[t].at[:, pl.ds(c0, 128)].at[midx_v.at[s]],
            buf.at[s], sems[s])

    n = len(_GROUPS)
    prep(0, 0)
    pend = {0: fire(0, 0)}
    for g in range(n):
        s = g & 1
        if g + 1 < n:
            prep(g + 1, 1 - s)
            pend[g + 1] = fire(g + 1, 1 - s)
        pend[g].wait()
        t, sl = _GROUPS[g]
        for i in range(_GK // _L):
            k = i * _L + lanes
            vals = plsc.load_gather(
                buf.at[s], [k, lane_v[s, pl.ds(i * _L, _L)]])
            pos = (t * _RPW + sl + 128 * i) + 8 * lanes
            plsc.store_scatter(vals_v, [pos], vals)
    for t in range(7):
        pltpu.sync_copy(vals_v.at[pl.ds(t * _RPW, _RPW)],
                        out_hbm.at[pl.ds(t * _R + base, _RPW)])


@functools.lru_cache(maxsize=None)
def _sc_extract():
    return pl.kernel(
        _sc_extract_body,
        out_type=jax.ShapeDtypeStruct((_NW * _FPW,), jnp.float32),
        mesh=plsc.VectorSubcoreMesh(core_axis_name="c", subcore_axis_name="s",
                                    num_cores=_NC, num_subcores=_NS),
        scratch_types=[
            pltpu.VMEM((7, 2, 128), jnp.int32),
            pltpu.VMEM((_FPW,), jnp.float32),
            pltpu.VMEM((2, _GK), jnp.int32),
            pltpu.VMEM((2, _GK), jnp.int32),
            pltpu.VMEM((2, _GK, 128), jnp.float32),
            pltpu.SemaphoreType.DMA,
            pltpu.SemaphoreType.DMA,
        ],
        compiler_params=pltpu.CompilerParams(use_tc_tiling_on_sc=True,
                                             needs_layout_passes=False),
    )


_LN2 = 0.6931471805599453
_CNEG = -(1.0 - _ALPHA) * 0.25


_CH = 16


def _fneg_sum(ref):
    n = ref.shape[2]
    acc = jnp.zeros((8, n), jnp.float32)
    for c in range(ref.shape[1] // _CH):
        x = ref[0, c * _CH:(c + 1) * _CH, :]
        t = jnp.tanh(0.5 * x)
        s = 1.0 + t
        v = (jnp.log(1.0 - t) - _LN2) * s * s
        acc += v.reshape(_CH // 8, 8, n).sum(axis=0)
    return jnp.sum(acc)


def _dense_body(e0, e1, e2, s0, s1, s2, s3, pp, tp, out):
    b = pl.program_id(0)
    qc = pl.program_id(1)

    @pl.when((b == 0) & (qc == 0))
    def _init():
        for t in range(8):
            out[0, t] = 0.0

    @pl.when(qc == 0)
    def _point():
        out[0, 0] += jnp.sum(jnp.abs(pp[...] - tp[...]))

    refs = (e0, e1, e2, s0, s1, s2, s3)
    for t in range(7):
        out[0, t + 1] += _fneg_sum(refs[t])


_QCH = 1
_dense_call = pl.pallas_call(
    _dense_body,
    grid=(_B, _QCH),
    in_specs=[
        pl.BlockSpec((1, _Q // _QCH, _Q), lambda b, qc: (b, qc, 0)),
        pl.BlockSpec((1, _Q // _QCH, _Q), lambda b, qc: (b, qc, 0)),
        pl.BlockSpec((1, _Q // _QCH, _Q), lambda b, qc: (b, qc, 0)),
        pl.BlockSpec((1, _Q // _QCH, _C), lambda b, qc: (b, qc, 0)),
        pl.BlockSpec((1, _Q // _QCH, _C), lambda b, qc: (b, qc, 0)),
        pl.BlockSpec((1, _Q // _QCH, _C), lambda b, qc: (b, qc, 0)),
        pl.BlockSpec((1, _Q // _QCH, _C), lambda b, qc: (b, qc, 0)),
        pl.BlockSpec((1, _Q, 2), lambda b, qc: (b, 0, 0)),
        pl.BlockSpec((1, _Q, 2), lambda b, qc: (b, 0, 0)),
    ],
    out_specs=pl.BlockSpec((1, 8), lambda b, qc: (0, 0),
                           memory_space=pltpu.SMEM),
    out_shape=jax.ShapeDtypeStruct((1, 8), jnp.float32),
    compiler_params=pltpu.CompilerParams(
        dimension_semantics=("arbitrary", "arbitrary")),
)


def _combine_body(part, g_ref, out):
    g = g_ref[...]
    th = jnp.tanh(0.5 * g)
    p = 0.5 * (1.0 + th)
    q = 0.5 * (1.0 - th)
    corr = (1.0 - _ALPHA) * jnp.log(q) * p * p - _ALPHA * jnp.log(p) * q * q
    out[0, 0] = part[0, 0] * _INV
    for t in range(7):
        s = jnp.sum(corr[t * 64:(t + 1) * 64, :])
        out[0, t + 1] = (_CNEG * part[0, t + 1] + s) * _INV


_combine_call = pl.pallas_call(
    _combine_body,
    in_specs=[
        pl.BlockSpec(memory_space=pltpu.SMEM),
        pl.BlockSpec(memory_space=pltpu.MemorySpace.VMEM),
    ],
    out_specs=pl.BlockSpec(memory_space=pltpu.SMEM),
    out_shape=jax.ShapeDtypeStruct((1, 8), jnp.float32),
)


def kernel(pred_points, pred_edges, pred_last_edges, pred_this_edges,
           pred_semantic_left_up, pred_semantic_right_up,
           pred_semantic_right_down, pred_semantic_left_down,
           target_points, edges_idx, last_edges_idx, this_edges_idx,
           sem_lu_idx, sem_ru_idx, sem_rd_idx, sem_ld_idx):
    dense = (pred_edges, pred_last_edges, pred_this_edges,
             pred_semantic_left_up, pred_semantic_right_up,
             pred_semantic_right_down, pred_semantic_left_down)
    idxs = (edges_idx, last_edges_idx, this_edges_idx,
            sem_lu_idx, sem_ru_idx, sem_rd_idx, sem_ld_idx)

    rows2d = [d.reshape(_R, n) for d, n in zip(dense, _SIZES)]
    gflat = _sc_extract()(*rows2d, *idxs)
    g = gflat.reshape(448, 128)

    part = _dense_call(*dense, target_points, pred_points)
    out = _combine_call(part, g)
    return out.reshape(8)


# speedup vs baseline: 3.8519x; 1.0498x over previous
import functools

import jax
import jax.numpy as jnp
from jax import lax
from jax.experimental import pallas as pl
from jax.experimental.pallas import tpu as pltpu
from jax.experimental.pallas import tpu_sc as plsc

_B, _Q, _C = 8, 1024, 512
_ALPHA = 0.25
_R = _B * _Q
_INV = 1.0 / float(_R)
_SIZES = (_Q, _Q, _Q, _C, _C, _C, _C)

_NC, _NS, _L = 2, 16, 16
_NW = _NC * _NS
_RPW = _R // _NW
_FPW = 7 * _RPW
_GK = _RPW // 8
_GROUPS = tuple((t, sl) for t in range(7) for sl in range(8))


def _sc_extract_body(e0, e1, e2, s0, s1, s2, s3,
                     i0, i1, i2, i3, i4, i5, i6, out_hbm,
                     idxb, vals_v, midx_v, lane_v, buf, sem0, sem1):
    data = (e0, e1, e2, s0, s1, s2, s3)
    idxs = (i0, i1, i2, i3, i4, i5, i6)
    sems = (sem0, sem1)
    w = lax.axis_index("s") * _NC + lax.axis_index("c")
    b = w // 4
    qq = w % 4
    base = w * _RPW
    lanes = lax.iota(jnp.int32, _L)

    ih = [pltpu.async_copy(
              idxs[t].at[:, pl.ds(b * 128, 128)].at[2 * qq + j],
              idxb.at[t].at[j], sems[j])
          for t in range(7) for j in range(2)]
    idx_ready = [False] * 7

    def prep(g, s):
        t, sl = _GROUPS[g]
        if not idx_ready[t]:
            ih[2 * t].wait()
            ih[2 * t + 1].wait()
            idx_ready[t] = True
        for i in range(_GK // _L):
            cols = plsc.load_gather(idxb.at[t].at[i], [sl + 8 * lanes])
            k = i * _L + lanes
            ct = cols >> 7
            if _SIZES[t] == _Q:
                m = base + 8 * k + ct
            else:
                m = base + 8 * k + 2 * ct + (sl >> 2)
            midx_v[s, pl.ds(i * _L, _L)] = m
            lane_v[s, pl.ds(i * _L, _L)] = cols & 127

    def fire(g, s):
        t, sl = _GROUPS[g]
        c0 = (sl if _SIZES[t] == _Q else (sl & 3)) * 128
        return pltpu.async_copy(
            data[t].at[:, pl.ds(c0, 128)].at[midx_v.at[s]],
            buf.at[s], sems[s])

    n = len(_GROUPS)
    prep(0, 0)
    pend = {0: fire(0, 0)}
    for g in range(n):
        s = g & 1
        if g + 1 < n:
            prep(g + 1, 1 - s)
            pend[g + 1] = fire(g + 1, 1 - s)
        pend[g].wait()
        t, sl = _GROUPS[g]
        for i in range(_GK // _L):
            k = i * _L + lanes
            vals = plsc.load_gather(
                buf.at[s], [k, lane_v[s, pl.ds(i * _L, _L)]])
            pos = (t * _RPW + sl + 128 * i) + 8 * lanes
            plsc.store_scatter(vals_v, [pos], vals)
    for t in range(7):
        pltpu.sync_copy(vals_v.at[pl.ds(t * _RPW, _RPW)],
                        out_hbm.at[pl.ds(t * _R + base, _RPW)])


@functools.lru_cache(maxsize=None)
def _sc_extract():
    return pl.kernel(
        _sc_extract_body,
        out_type=jax.ShapeDtypeStruct((_NW * _FPW,), jnp.float32),
        mesh=plsc.VectorSubcoreMesh(core_axis_name="c", subcore_axis_name="s",
                                    num_cores=_NC, num_subcores=_NS),
        scratch_types=[
            pltpu.VMEM((7, 2, 128), jnp.int32),
            pltpu.VMEM((_FPW,), jnp.float32),
            pltpu.VMEM((2, _GK), jnp.int32),
            pltpu.VMEM((2, _GK), jnp.int32),
            pltpu.VMEM((2, _GK, 128), jnp.float32),
            pltpu.SemaphoreType.DMA,
            pltpu.SemaphoreType.DMA,
        ],
        compiler_params=pltpu.CompilerParams(use_tc_tiling_on_sc=True,
                                             needs_layout_passes=False),
    )


_LN2 = 0.6931471805599453
_CNEG = -(1.0 - _ALPHA) * 0.25


_CH = 16


def _fneg_sum(ref):
    n = ref.shape[2]
    acc = jnp.zeros((8, n), jnp.float32)
    for c in range(ref.shape[1] // _CH):
        x = ref[0, c * _CH:(c + 1) * _CH, :]
        t = jnp.tanh(0.5 * x)
        s = 1.0 + t
        v = (jnp.log(1.0 - t) - _LN2) * s * s
        acc += v.reshape(_CH // 8, 8, n).sum(axis=0)
    return jnp.sum(acc)


def _dense_body(e0, e1, e2, s0, s1, s2, s3, pp, tp, out):
    b = pl.program_id(0)
    qc = pl.program_id(1)

    @pl.when((b == 0) & (qc == 0))
    def _init():
        for t in range(8):
            out[0, t] = 0.0

    @pl.when(qc == 0)
    def _point():
        out[0, 0] += jnp.sum(jnp.abs(pp[...] - tp[...]))

    refs = (e0, e1, e2, s0, s1, s2, s3)
    for t in range(7):
        out[0, t + 1] += _fneg_sum(refs[t])


_QCH = 1
_dense_call = pl.pallas_call(
    _dense_body,
    grid=(_B, _QCH),
    in_specs=[
        pl.BlockSpec((1, _Q // _QCH, _Q), lambda b, qc: (b, qc, 0)),
        pl.BlockSpec((1, _Q // _QCH, _Q), lambda b, qc: (b, qc, 0)),
        pl.BlockSpec((1, _Q // _QCH, _Q), lambda b, qc: (b, qc, 0)),
        pl.BlockSpec((1, _Q // _QCH, _C), lambda b, qc: (b, qc, 0)),
        pl.BlockSpec((1, _Q // _QCH, _C), lambda b, qc: (b, qc, 0)),
        pl.BlockSpec((1, _Q // _QCH, _C), lambda b, qc: (b, qc, 0)),
        pl.BlockSpec((1, _Q // _QCH, _C), lambda b, qc: (b, qc, 0)),
        pl.BlockSpec((1, 1, 2 * _Q), lambda b, qc: (b, 0, 0)),
        pl.BlockSpec((1, 1, 2 * _Q), lambda b, qc: (b, 0, 0)),
    ],
    out_specs=pl.BlockSpec((1, 8), lambda b, qc: (0, 0),
                           memory_space=pltpu.SMEM),
    out_shape=jax.ShapeDtypeStruct((1, 8), jnp.float32),
    compiler_params=pltpu.CompilerParams(
        dimension_semantics=("arbitrary", "arbitrary")),
)


def _combine_body(part, g_ref, out):
    g = g_ref[...]
    th = jnp.tanh(0.5 * g)
    p = 0.5 * (1.0 + th)
    q = 0.5 * (1.0 - th)
    corr = (1.0 - _ALPHA) * jnp.log(q) * p * p - _ALPHA * jnp.log(p) * q * q
    out[0, 0] = part[0, 0] * _INV
    for t in range(7):
        s = jnp.sum(corr[t * 64:(t + 1) * 64, :])
        out[0, t + 1] = (_CNEG * part[0, t + 1] + s) * _INV


_combine_call = pl.pallas_call(
    _combine_body,
    in_specs=[
        pl.BlockSpec(memory_space=pltpu.SMEM),
        pl.BlockSpec(memory_space=pltpu.MemorySpace.VMEM),
    ],
    out_specs=pl.BlockSpec(memory_space=pltpu.SMEM),
    out_shape=jax.ShapeDtypeStruct((1, 8), jnp.float32),
)


def kernel(pred_points, pred_edges, pred_last_edges, pred_this_edges,
           pred_semantic_left_up, pred_semantic_right_up,
           pred_semantic_right_down, pred_semantic_left_down,
           target_points, edges_idx, last_edges_idx, this_edges_idx,
           sem_lu_idx, sem_ru_idx, sem_rd_idx, sem_ld_idx):
    dense = (pred_edges, pred_last_edges, pred_this_edges,
             pred_semantic_left_up, pred_semantic_right_up,
             pred_semantic_right_down, pred_semantic_left_down)
    idxs = (edges_idx, last_edges_idx, this_edges_idx,
            sem_lu_idx, sem_ru_idx, sem_rd_idx, sem_ld_idx)

    rows2d = [d.reshape(_R, n) for d, n in zip(dense, _SIZES)]
    gflat = _sc_extract()(*rows2d, *idxs)
    g = gflat.reshape(448, 128)

    part = _dense_call(*dense, target_points.reshape(_B, 1, 2 * _Q),
                       pred_points.reshape(_B, 1, 2 * _Q))
    out = _combine_call(part, g)
    return out.reshape(8)
